# baseline (speedup 1.0000x reference)
import functools

import jax
import jax.numpy as jnp
from jax import lax
from jax.experimental import pallas as pl
from jax.experimental.pallas import tpu as pltpu
from jax.experimental.pallas import tpu_sc as plsc

ALPHA = 0.5

_ROWS = 8192
_COLS = 1024
_N = _ROWS * _COLS

_NC, _NS, _L = 2, 16, 16
_NW = _NC * _NS
_CHUNK = 16384
_U = 8

_SC_ROWS = _ROWS
_SC_N = _SC_ROWS * _COLS
_PER_W = _SC_N // _NW
_NCHUNK = _PER_W // _CHUNK

_sc_mesh = plsc.VectorSubcoreMesh(
    core_axis_name="c", subcore_axis_name="s",
    num_cores=_NC, num_subcores=_NS)


def _tree_sum(xs):
    while len(xs) > 1:
        xs = [a + b for a, b in zip(xs[::2], xs[1::2])]
    return xs[0]


@functools.partial(
    pl.kernel,
    out_type=jax.ShapeDtypeStruct((_NW, 4, _L), jnp.float32),
    mesh=_sc_mesh,
    scratch_types=[
        pltpu.VMEM((2, _CHUNK), jnp.float32),
        pltpu.VMEM((2, _CHUNK), jnp.float32),
        pltpu.VMEM((2, _CHUNK), jnp.int32),
        pltpu.VMEM((4, _L), jnp.float32),
        pltpu.SemaphoreType.DMA,
        pltpu.SemaphoreType.DMA,
    ],
)
def _sc_reduce(p_hbm, t_hbm, m_hbm, out_hbm, pbuf, tbuf, mbuf, obuf,
               sem0, sem1):
    wid = lax.axis_index("s") * _NC + lax.axis_index("c")
    base = wid * _PER_W
    sems = (sem0, sem1)

    def start(g):
        b = g % 2
        s = base + g * _CHUNK
        return (
            pltpu.async_copy(p_hbm.at[pl.ds(s, _CHUNK)], pbuf.at[b], sems[b]),
            pltpu.async_copy(t_hbm.at[pl.ds(s, _CHUNK)], tbuf.at[b], sems[b]),
            pltpu.async_copy(m_hbm.at[pl.ds(s, _CHUNK)], mbuf.at[b], sems[b]),
        )

    def chunk_reduce(b, accs):
        def body(i, carry):
            a1, a2, a3, a4 = carry
            s1s, s2s, s3s, s4s = [], [], [], []
            for u in range(_U):
                off = pl.ds(pl.multiple_of(i + u * _L, _L), _L)
                p = pbuf[b, off]
                t = tbuf[b, off]
                m = mbuf[b, off].astype(jnp.float32)
                ad = jnp.abs(t - p)
                nz = jnp.where(jnp.abs(t) > 1.0, m, 0.0)
                s1s.append(ad * m)
                s2s.append(m)
                s3s.append(ad * nz)
                s4s.append(nz)
            return (a1 + _tree_sum(s1s), a2 + _tree_sum(s2s),
                    a3 + _tree_sum(s3s), a4 + _tree_sum(s4s))
        return plsc.parallel_loop(0, _CHUNK, step=_U * _L, carry=accs)(body)

    z = jnp.zeros((_L,), jnp.float32)
    accs = (z, z, z, z)
    descs = [None, None]
    descs[0] = start(0)
    for g in range(_NCHUNK):
        b = g % 2
        if g + 1 < _NCHUNK:
            descs[1 - b] = start(g + 1)
        for d in descs[b]:
            d.wait()
        accs = chunk_reduce(b, accs)

    obuf[0] = accs[0]
    obuf[1] = accs[1]
    obuf[2] = accs[2]
    obuf[3] = accs[3]
    pltpu.sync_copy(obuf, out_hbm.at[wid])


_TC_BLK = 1024


def _tc_body(p_ref, t_ref, m_ref, out_ref, acc_ref):
    i = pl.program_id(0)
    p = p_ref[...]
    t = t_ref[...]
    maskf = m_ref[...].astype(jnp.float32)
    absdiff = jnp.abs(t - p)
    nzf = jnp.where(jnp.abs(t) > 1.0, maskf, 0.0)
    s1 = jnp.sum(absdiff * maskf)
    c1 = jnp.sum(maskf)
    s2 = jnp.sum(absdiff * nzf)
    c2 = jnp.sum(nzf)

    @pl.when(i == 0)
    def _init():
        acc_ref[0] = s1
        acc_ref[1] = c1
        acc_ref[2] = s2
        acc_ref[3] = c2

    @pl.when(i > 0)
    def _accum():
        acc_ref[0] += s1
        acc_ref[1] += c1
        acc_ref[2] += s2
        acc_ref[3] += c2

    @pl.when(i == pl.num_programs(0) - 1)
    def _finish():
        out_ref[0] = acc_ref[0]
        out_ref[1] = acc_ref[1]
        out_ref[2] = acc_ref[2]
        out_ref[3] = acc_ref[3]


def _tc_reduce(p, t, m):
    rows = p.shape[0]
    grid = rows // _TC_BLK
    return pl.pallas_call(
        _tc_body,
        grid=(grid,),
        in_specs=[
            pl.BlockSpec((_TC_BLK, _COLS), lambda i: (i, 0)),
            pl.BlockSpec((_TC_BLK, _COLS), lambda i: (i, 0)),
            pl.BlockSpec((_TC_BLK, _COLS), lambda i: (i, 0)),
        ],
        out_specs=pl.BlockSpec(memory_space=pltpu.SMEM),
        out_shape=jax.ShapeDtypeStruct((4,), jnp.float32),
        scratch_shapes=[pltpu.SMEM((4,), jnp.float32)],
    )(p, t, m)


def kernel(pred, masked_input, mask_label, target, masked_only_input):
    del masked_input, masked_only_input
    p = pred.reshape(_ROWS, _COLS)
    t = target.reshape(_ROWS, _COLS)
    m = mask_label.reshape(_ROWS, _COLS)

    s = jnp.zeros((4,), jnp.float32)
    if _SC_ROWS > 0:
        parts = _sc_reduce(p[:_SC_ROWS].reshape(_SC_N),
                           t[:_SC_ROWS].reshape(_SC_N),
                           m[:_SC_ROWS].reshape(_SC_N))
        s = s + jnp.sum(parts, axis=(0, 2))
    if _SC_ROWS < _ROWS:
        s = s + _tc_reduce(p[_SC_ROWS:], t[_SC_ROWS:], m[_SC_ROWS:])
    return s[0] / s[1] + ALPHA * s[2] / s[3]
